# baseline (speedup 1.0000x reference)
import functools

import jax
import jax.numpy as jnp
from jax import lax
from jax.experimental import pallas as pl
from jax.experimental.pallas import tpu as pltpu
from jax.experimental.pallas import tpu_sc as plsc

N_GENES = 100000
N_LATENT = 16
N_COMP = 32
B = 256
G_OI = 500

G_PAD = 512

_INFO = plsc.get_sparse_core_info()
_NC, _NS = _INFO.num_cores, _INFO.num_subcores
_NW = _NC * _NS
_B_PER_W = G_PAD // _NW

_sc_mesh = plsc.VectorSubcoreMesh(core_axis_name="c", subcore_axis_name="s")


@functools.partial(
    pl.kernel,
    mesh=_sc_mesh,
    out_type=jax.ShapeDtypeStruct((G_PAD, N_LATENT, N_COMP), jnp.float32),
    scratch_types=[
        pltpu.VMEM((_B_PER_W,), jnp.int32),
        pltpu.VMEM((_B_PER_W, N_LATENT, N_COMP), jnp.float32),
        pltpu.SemaphoreType.DMA,
    ],
)
def _sc_gather(idx_hbm, table_hbm, out_hbm, idx_v, rows_v, sem):
    wid = lax.axis_index("s") * _NC + lax.axis_index("c")
    base = wid * _B_PER_W
    pltpu.sync_copy(idx_hbm.at[pl.ds(base, _B_PER_W)], idx_v)
    idx_vec = idx_v[...]
    handles = []
    for j in range(_B_PER_W):
        row = idx_vec[j]
        handles.append(pltpu.async_copy(table_hbm.at[row], rows_v.at[j], sem))
    for h in handles:
        h.wait()
    pltpu.sync_copy(rows_v, out_hbm.at[pl.ds(base, _B_PER_W)])


def _logit_body(lat_ref, g_ref, out_ref):
    out_ref[...] = lax.dot_general(
        lat_ref[...], g_ref[...], (((1,), (1,)), ((), ())),
        preferred_element_type=jnp.float32)


def _rho_body(lat_ref, w_ref, out_ref):
    out_ref[...] = lax.dot_general(
        lat_ref[...], w_ref[...], (((1,), (1,)), ((), ())),
        preferred_element_type=jnp.float32)


_RHO_CHUNK = 4096


def kernel(latent, genes_oi, logit_weight_table, rho_weight_table):
    idx = jnp.pad(genes_oi, (0, G_PAD - G_OI))
    gathered = _sc_gather(idx, logit_weight_table)

    logit = pl.pallas_call(
        _logit_body,
        grid=(4,),
        in_specs=[
            pl.BlockSpec((B, N_LATENT), lambda i: (0, 0)),
            pl.BlockSpec((128, N_LATENT, N_COMP), lambda i: (i, 0, 0)),
        ],
        out_specs=pl.BlockSpec((B, 128, N_COMP), lambda i: (0, i, 0)),
        out_shape=jax.ShapeDtypeStruct((B, G_OI, N_COMP), jnp.float32),
    )(latent, gathered)

    n_chunks = pl.cdiv(N_GENES, _RHO_CHUNK)
    rho = pl.pallas_call(
        _rho_body,
        grid=(n_chunks,),
        in_specs=[
            pl.BlockSpec((B, N_LATENT), lambda i: (0, 0)),
            pl.BlockSpec((_RHO_CHUNK, N_LATENT), lambda i: (i, 0)),
        ],
        out_specs=pl.BlockSpec((B, _RHO_CHUNK), lambda i: (0, i)),
        out_shape=jax.ShapeDtypeStruct((B, N_GENES), jnp.float32),
    )(latent, rho_weight_table)
    return (logit, rho)


# speedup vs baseline: 1.8727x; 1.8727x over previous
import functools

import jax
import jax.numpy as jnp
from jax import lax
from jax.experimental import pallas as pl
from jax.experimental.pallas import tpu as pltpu
from jax.experimental.pallas import tpu_sc as plsc

N_GENES = 100000
N_LATENT = 16
N_COMP = 32
B = 256
G_OI = 500

G_PAD = 512

_INFO = plsc.get_sparse_core_info()
_NC, _NS = _INFO.num_cores, _INFO.num_subcores
_NW = _NC * _NS
_B_PER_W = G_PAD // _NW

_sc_mesh = plsc.VectorSubcoreMesh(core_axis_name="c", subcore_axis_name="s")


_ROW = N_LATENT * N_COMP


@functools.partial(
    pl.kernel,
    mesh=_sc_mesh,
    out_type=jax.ShapeDtypeStruct((G_PAD, _ROW), jnp.float32),
    scratch_types=[
        pltpu.VMEM((_B_PER_W,), jnp.int32),
        pltpu.VMEM((_B_PER_W, _ROW), jnp.float32),
        pltpu.SemaphoreType.DMA,
    ],
)
def _sc_gather(idx_hbm, table_hbm, out_hbm, idx_v, rows_v, sem):
    wid = lax.axis_index("s") * _NC + lax.axis_index("c")
    base = wid * _B_PER_W
    pltpu.sync_copy(idx_hbm.at[pl.ds(base, _B_PER_W)], idx_v)
    pltpu.async_copy(table_hbm.at[idx_v], rows_v, sem).wait()
    pltpu.sync_copy(rows_v, out_hbm.at[pl.ds(base, _B_PER_W)])


def _logit_body(lat_ref, g_ref, out_ref):
    out_ref[...] = lax.dot_general(
        lat_ref[...], g_ref[...], (((1,), (1,)), ((), ())),
        preferred_element_type=jnp.float32)


def _rho_body(lat_ref, w_ref, out_ref):
    out_ref[...] = lax.dot_general(
        lat_ref[...], w_ref[...], (((1,), (1,)), ((), ())),
        preferred_element_type=jnp.float32)


_RHO_CHUNK = 4096


def kernel(latent, genes_oi, logit_weight_table, rho_weight_table):
    idx = jnp.pad(genes_oi, (0, G_PAD - G_OI))
    table2 = logit_weight_table.reshape(N_GENES, _ROW)
    gathered = _sc_gather(idx, table2).reshape(G_PAD, N_LATENT, N_COMP)

    logit = pl.pallas_call(
        _logit_body,
        grid=(4,),
        in_specs=[
            pl.BlockSpec((B, N_LATENT), lambda i: (0, 0)),
            pl.BlockSpec((128, N_LATENT, N_COMP), lambda i: (i, 0, 0)),
        ],
        out_specs=pl.BlockSpec((B, 128, N_COMP), lambda i: (0, i, 0)),
        out_shape=jax.ShapeDtypeStruct((B, G_OI, N_COMP), jnp.float32),
    )(latent, gathered)

    n_chunks = pl.cdiv(N_GENES, _RHO_CHUNK)
    rho = pl.pallas_call(
        _rho_body,
        grid=(n_chunks,),
        in_specs=[
            pl.BlockSpec((B, N_LATENT), lambda i: (0, 0)),
            pl.BlockSpec((_RHO_CHUNK, N_LATENT), lambda i: (i, 0)),
        ],
        out_specs=pl.BlockSpec((B, _RHO_CHUNK), lambda i: (0, i)),
        out_shape=jax.ShapeDtypeStruct((B, N_GENES), jnp.float32),
    )(latent, rho_weight_table)
    return (logit, rho)


# speedup vs baseline: 2.3049x; 1.2308x over previous
import functools

import jax
import jax.numpy as jnp
from jax import lax
from jax.experimental import pallas as pl
from jax.experimental.pallas import tpu as pltpu
from jax.experimental.pallas import tpu_sc as plsc

N_GENES = 100000
N_LATENT = 16
N_COMP = 32
B = 256
G_OI = 500

G_PAD = 512
_ROW = N_LATENT * N_COMP

_INFO = plsc.get_sparse_core_info()
_NC, _NS = _INFO.num_cores, _INFO.num_subcores
_NW = _NC * _NS
_D_PER_W = _ROW // _NW

_sc_mesh = plsc.VectorSubcoreMesh(core_axis_name="c", subcore_axis_name="s")


_G_PER_W = G_PAD // _NW
_NSTR = _G_PER_W * 4


@functools.partial(
    pl.kernel,
    mesh=_sc_mesh,
    out_type=jax.ShapeDtypeStruct((G_PAD * 4, 128), jnp.float32),
    scratch_types=[
        pltpu.VMEM((_G_PER_W,), jnp.int32),
        pltpu.VMEM((_NSTR, 128), jnp.int32),
        pltpu.VMEM((_NSTR, 128), jnp.float32),
        pltpu.SemaphoreType.DMA,
    ],
)
def _sc_gather(idx_hbm, t1_hbm, out_hbm, idx_v, fidx_v, cols_v, sem):
    w = lax.axis_index("s") * _NC + lax.axis_index("c")
    base = w * _G_PER_W
    pltpu.sync_copy(idx_hbm.at[pl.ds(base, _G_PER_W)], idx_v)
    idx_vec = idx_v[...]
    lane = lax.iota(jnp.int32, 16)
    kv = [(lane + 16 * v) * N_GENES for v in range(_ROW // 16)]
    for j in range(_G_PER_W):
        g = idx_vec[j]
        for v in range(_ROW // 16):
            fidx_v[j * 4 + v // 8, pl.ds((v % 8) * 16, 16)] = kv[v] + g
    handles = []
    for s in range(_NSTR):
        handles.append(
            pltpu.async_copy(t1_hbm.at[fidx_v.at[s]], cols_v.at[s], sem))
    for h in handles:
        h.wait()
    pltpu.sync_copy(cols_v, out_hbm.at[pl.ds(w * _NSTR, _NSTR)])


def _logit_body(lat_ref, w_ref, out_ref):
    out_ref[...] = lax.dot_general(
        w_ref[...], lat_ref[...], (((1,), (0,)), ((), ())),
        preferred_element_type=jnp.float32)


def _rho_body(lat_ref, w_ref, out_ref):
    out_ref[...] = lax.dot_general(
        w_ref[...], lat_ref[...], (((0,), (0,)), ((), ())),
        preferred_element_type=jnp.float32)


_RHO_CHUNK = 4096


def kernel(latent, genes_oi, logit_weight_table, rho_weight_table):
    latT = latent.T
    wT = rho_weight_table.T
    t1 = jnp.transpose(logit_weight_table, (1, 2, 0)).reshape(_ROW * N_GENES)
    idx = jnp.pad(genes_oi, (0, G_PAD - G_OI))

    w_sc = _sc_gather(idx, t1).reshape(G_PAD, N_LATENT, N_COMP)

    logit_t = pl.pallas_call(
        _logit_body,
        grid=(4,),
        in_specs=[
            pl.BlockSpec((N_LATENT, B), lambda i: (0, 0)),
            pl.BlockSpec((128, N_LATENT, N_COMP), lambda i: (i, 0, 0)),
        ],
        out_specs=pl.BlockSpec((128, N_COMP, B), lambda i: (i, 0, 0)),
        out_shape=jax.ShapeDtypeStruct((G_OI, N_COMP, B), jnp.float32),
    )(latT, w_sc)

    n_chunks = pl.cdiv(N_GENES, _RHO_CHUNK)
    rho_t = pl.pallas_call(
        _rho_body,
        grid=(n_chunks,),
        in_specs=[
            pl.BlockSpec((N_LATENT, B), lambda i: (0, 0)),
            pl.BlockSpec((N_LATENT, _RHO_CHUNK), lambda i: (0, i)),
        ],
        out_specs=pl.BlockSpec((_RHO_CHUNK, B), lambda i: (i, 0)),
        out_shape=jax.ShapeDtypeStruct((N_GENES, B), jnp.float32),
    )(latT, wT)

    return (jnp.transpose(logit_t, (2, 0, 1)), rho_t.T)


# speedup vs baseline: 5.1379x; 2.2291x over previous
import jax
import jax.numpy as jnp
from jax import lax
from jax.experimental import pallas as pl
from jax.experimental.pallas import tpu as pltpu

N_GENES = 100000
N_LATENT = 16
N_COMP = 32
B = 256
G_OI = 500

_ROW = N_LATENT * N_COMP
_G_STEP = 4
_RHO_CHUNK = 4096


def _logit_body(idx_ref, lat2_ref, b0, b1, b2, b3, out_ref):
    i = pl.program_id(0)
    kidx = lax.broadcasted_iota(jnp.int32, (N_COMP, _ROW), 1)
    cidx = lax.broadcasted_iota(jnp.int32, (N_COMP, _ROW), 0)
    amask = (kidx % N_COMP) == cidx
    lane_iota = lax.broadcasted_iota(jnp.int32, (1, 128), 1)
    lat2 = lat2_ref[...]
    for j, blk in enumerate((b0, b1, b2, b3)):
        lane = idx_ref[_G_STEP * i + j] & 127
        onehot = (lane_iota == lane).astype(jnp.float32)
        vrow = lax.dot_general(
            onehot, blk[...], (((1,), (1,)), ((), ())),
            preferred_element_type=jnp.float32)
        u = jnp.where(amask, vrow, 0.0)
        y = lax.dot_general(
            u, lat2, (((1,), (0,)), ((), ())),
            preferred_element_type=jnp.float32)
        out_ref[j] = y


def _rho_body(lat_ref, w_ref, out_ref):
    out_ref[...] = lax.dot_general(
        w_ref[...], lat_ref[...], (((0,), (0,)), ((), ())),
        preferred_element_type=jnp.float32)


def kernel(latent, genes_oi, logit_weight_table, rho_weight_table):
    latT = latent.T
    wT = rho_weight_table.T
    t2 = jnp.transpose(logit_weight_table, (1, 2, 0)).reshape(_ROW, N_GENES)
    lat2 = jnp.repeat(latT, N_COMP, axis=0)

    tbl_spec = [
        pl.BlockSpec(
            (_ROW, 128),
            (lambda j: lambda i, idx_ref: (0, idx_ref[_G_STEP * i + j] >> 7))(j),
        )
        for j in range(_G_STEP)
    ]
    logit_t = pl.pallas_call(
        _logit_body,
        grid_spec=pltpu.PrefetchScalarGridSpec(
            num_scalar_prefetch=1,
            grid=(G_OI // _G_STEP,),
            in_specs=[pl.BlockSpec((_ROW, B), lambda i, idx_ref: (0, 0))]
            + tbl_spec,
            out_specs=pl.BlockSpec(
                (_G_STEP, N_COMP, B), lambda i, idx_ref: (i, 0, 0)),
        ),
        out_shape=jax.ShapeDtypeStruct((G_OI, N_COMP, B), jnp.float32),
    )(genes_oi, lat2, t2, t2, t2, t2)

    n_chunks = pl.cdiv(N_GENES, _RHO_CHUNK)
    rho_t = pl.pallas_call(
        _rho_body,
        grid=(n_chunks,),
        in_specs=[
            pl.BlockSpec((N_LATENT, B), lambda i: (0, 0)),
            pl.BlockSpec((N_LATENT, _RHO_CHUNK), lambda i: (0, i)),
        ],
        out_specs=pl.BlockSpec((_RHO_CHUNK, B), lambda i: (i, 0)),
        out_shape=jax.ShapeDtypeStruct((N_GENES, B), jnp.float32),
    )(latT, wT)

    return (jnp.transpose(logit_t, (2, 0, 1)), rho_t.T)


# speedup vs baseline: 6.7264x; 1.3092x over previous
import jax
import jax.numpy as jnp
from jax import lax
from jax.experimental import pallas as pl
from jax.experimental.pallas import tpu as pltpu

N_GENES = 100000
N_LATENT = 16
N_COMP = 32
B = 256
G_OI = 500

_ROW = N_LATENT * N_COMP
_G_STEP = 8
_G_PAD = 504
_RHO_CHUNK = 4096


def _gather_body(idx_ref, b0, b1, b2, b3, b4, b5, b6, b7, out_ref):
    i = pl.program_id(0)
    lane_iota = lax.broadcasted_iota(jnp.int32, (1, 128), 1)
    for j, blk in enumerate((b0, b1, b2, b3, b4, b5, b6, b7)):
        lane = idx_ref[_G_STEP * i + j] & 127
        onehot = (lane_iota == lane).astype(jnp.float32)
        out_ref[j, :] = lax.dot_general(
            onehot, blk[...], (((1,), (1,)), ((), ())),
            preferred_element_type=jnp.float32)[0]


def _logit_body(lat_ref, w_ref, out_ref):
    out_ref[...] = lax.dot_general(
        w_ref[...], lat_ref[...], (((1,), (0,)), ((), ())),
        preferred_element_type=jnp.float32)


def _rho_body(lat_ref, w_ref, out_ref):
    out_ref[...] = lax.dot_general(
        w_ref[...], lat_ref[...], (((0,), (0,)), ((), ())),
        preferred_element_type=jnp.float32)


def kernel(latent, genes_oi, logit_weight_table, rho_weight_table):
    latT = latent.T
    wT = rho_weight_table.T
    t2 = jnp.transpose(logit_weight_table, (1, 2, 0)).reshape(_ROW, N_GENES)
    idx = jnp.pad(genes_oi, (0, _G_PAD - G_OI))

    tbl_spec = [
        pl.BlockSpec(
            (_ROW, 128),
            (lambda j: lambda i, idx_ref: (0, idx_ref[_G_STEP * i + j] >> 7))(j),
        )
        for j in range(_G_STEP)
    ]
    w_rows = pl.pallas_call(
        _gather_body,
        grid_spec=pltpu.PrefetchScalarGridSpec(
            num_scalar_prefetch=1,
            grid=(_G_PAD // _G_STEP,),
            in_specs=tbl_spec,
            out_specs=pl.BlockSpec((_G_STEP, _ROW), lambda i, idx_ref: (i, 0)),
        ),
        out_shape=jax.ShapeDtypeStruct((G_OI, _ROW), jnp.float32),
    )(idx, t2, t2, t2, t2, t2, t2, t2, t2)
    w3 = w_rows.reshape(G_OI, N_LATENT, N_COMP)

    logit_t = pl.pallas_call(
        _logit_body,
        grid=(4,),
        in_specs=[
            pl.BlockSpec((N_LATENT, B), lambda i: (0, 0)),
            pl.BlockSpec((128, N_LATENT, N_COMP), lambda i: (i, 0, 0)),
        ],
        out_specs=pl.BlockSpec((128, N_COMP, B), lambda i: (i, 0, 0)),
        out_shape=jax.ShapeDtypeStruct((G_OI, N_COMP, B), jnp.float32),
    )(latT, w3)

    n_chunks = pl.cdiv(N_GENES, _RHO_CHUNK)
    rho_t = pl.pallas_call(
        _rho_body,
        grid=(n_chunks,),
        in_specs=[
            pl.BlockSpec((N_LATENT, B), lambda i: (0, 0)),
            pl.BlockSpec((N_LATENT, _RHO_CHUNK), lambda i: (0, i)),
        ],
        out_specs=pl.BlockSpec((_RHO_CHUNK, B), lambda i: (i, 0)),
        out_shape=jax.ShapeDtypeStruct((N_GENES, B), jnp.float32),
    )(latT, wT)

    return (jnp.transpose(logit_t, (2, 0, 1)), rho_t.T)


# speedup vs baseline: 7.4710x; 1.1107x over previous
import jax
import jax.numpy as jnp
from jax import lax
from jax.experimental import pallas as pl
from jax.experimental.pallas import tpu as pltpu

N_GENES = 100000
N_LATENT = 16
N_COMP = 32
B = 256
G_OI = 500

_ROW = N_LATENT * N_COMP
_G_STEP = 16
_G_PAD = 512
_RHO_CHUNK = 4096


def _gather_body(idx_ref, *refs):
    bs, out_ref = refs[:-1], refs[-1]
    i = pl.program_id(0)
    lane_iota = lax.broadcasted_iota(jnp.int32, (1, 128), 1)
    for j, blk in enumerate(bs):
        lane = idx_ref[_G_STEP * i + j] & 127
        onehot = (lane_iota == lane).astype(jnp.float32)
        out_ref[j, :] = lax.dot_general(
            onehot, blk[...], (((1,), (1,)), ((), ())),
            preferred_element_type=jnp.float32)[0]


def _logit_body(lat_ref, w_ref, out_ref):
    out_ref[...] = lax.dot_general(
        w_ref[...], lat_ref[...], (((1,), (0,)), ((), ())),
        preferred_element_type=jnp.float32)


def _rho_body(lat_ref, w_ref, out_ref):
    out_ref[...] = lax.dot_general(
        w_ref[...], lat_ref[...], (((0,), (0,)), ((), ())),
        preferred_element_type=jnp.float32)


def kernel(latent, genes_oi, logit_weight_table, rho_weight_table):
    latT = latent.T
    wT = rho_weight_table.T
    t2 = jnp.transpose(logit_weight_table, (1, 2, 0)).reshape(_ROW, N_GENES)
    idx = jnp.pad(genes_oi, (0, _G_PAD - G_OI))

    tbl_spec = [
        pl.BlockSpec(
            (_ROW, 128),
            (lambda j: lambda i, idx_ref: (0, idx_ref[_G_STEP * i + j] >> 7))(j),
        )
        for j in range(_G_STEP)
    ]
    w_rows = pl.pallas_call(
        _gather_body,
        grid_spec=pltpu.PrefetchScalarGridSpec(
            num_scalar_prefetch=1,
            grid=(_G_PAD // _G_STEP,),
            in_specs=tbl_spec,
            out_specs=pl.BlockSpec((_G_STEP, _ROW), lambda i, idx_ref: (i, 0)),
        ),
        out_shape=jax.ShapeDtypeStruct((G_OI, _ROW), jnp.float32),
    )(idx, *([t2] * _G_STEP))
    w3 = w_rows.reshape(G_OI, N_LATENT, N_COMP)

    logit_t = pl.pallas_call(
        _logit_body,
        grid=(4,),
        in_specs=[
            pl.BlockSpec((N_LATENT, B), lambda i: (0, 0)),
            pl.BlockSpec((128, N_LATENT, N_COMP), lambda i: (i, 0, 0)),
        ],
        out_specs=pl.BlockSpec((128, N_COMP, B), lambda i: (i, 0, 0)),
        out_shape=jax.ShapeDtypeStruct((G_OI, N_COMP, B), jnp.float32),
    )(latT, w3)

    n_chunks = pl.cdiv(N_GENES, _RHO_CHUNK)
    rho_t = pl.pallas_call(
        _rho_body,
        grid=(n_chunks,),
        in_specs=[
            pl.BlockSpec((N_LATENT, B), lambda i: (0, 0)),
            pl.BlockSpec((N_LATENT, _RHO_CHUNK), lambda i: (0, i)),
        ],
        out_specs=pl.BlockSpec((_RHO_CHUNK, B), lambda i: (i, 0)),
        out_shape=jax.ShapeDtypeStruct((N_GENES, B), jnp.float32),
    )(latT, wT)

    return (jnp.transpose(logit_t, (2, 0, 1)), rho_t.T)


# speedup vs baseline: 8.4642x; 1.1330x over previous
import jax
import jax.numpy as jnp
from jax import lax
from jax.experimental import pallas as pl
from jax.experimental.pallas import tpu as pltpu

N_GENES = 100000
N_LATENT = 16
N_COMP = 32
B = 256
G_OI = 500

_ROW = N_LATENT * N_COMP
_G_STEP = 32
_G_PAD = 512
_RHO_CHUNK = 8192


def _gather_body(idx_ref, *refs):
    bs, out_ref = refs[:-1], refs[-1]
    i = pl.program_id(0)
    lane_iota = lax.broadcasted_iota(jnp.int32, (1, 128), 1)
    for j, blk in enumerate(bs):
        lane = idx_ref[_G_STEP * i + j] & 127
        onehot = (lane_iota == lane).astype(jnp.float32)
        out_ref[j, :] = lax.dot_general(
            onehot, blk[...], (((1,), (1,)), ((), ())),
            preferred_element_type=jnp.float32)[0]


def _logit_body(lat_ref, w_ref, out_ref):
    out_ref[...] = lax.dot_general(
        w_ref[...], lat_ref[...], (((1,), (0,)), ((), ())),
        preferred_element_type=jnp.float32)


def _rho_body(lat_ref, w_ref, out_ref):
    out_ref[...] = lax.dot_general(
        w_ref[...], lat_ref[...], (((0,), (0,)), ((), ())),
        preferred_element_type=jnp.float32)


def kernel(latent, genes_oi, logit_weight_table, rho_weight_table):
    latT = latent.T
    wT = rho_weight_table.T
    t2 = jnp.transpose(logit_weight_table, (1, 2, 0)).reshape(_ROW, N_GENES)
    idx = jnp.pad(genes_oi, (0, _G_PAD - G_OI))

    tbl_spec = [
        pl.BlockSpec(
            (_ROW, 128),
            (lambda j: lambda i, idx_ref: (0, idx_ref[_G_STEP * i + j] >> 7))(j),
        )
        for j in range(_G_STEP)
    ]
    w_rows = pl.pallas_call(
        _gather_body,
        grid_spec=pltpu.PrefetchScalarGridSpec(
            num_scalar_prefetch=1,
            grid=(_G_PAD // _G_STEP,),
            in_specs=tbl_spec,
            out_specs=pl.BlockSpec((_G_STEP, _ROW), lambda i, idx_ref: (i, 0)),
        ),
        out_shape=jax.ShapeDtypeStruct((G_OI, _ROW), jnp.float32),
    )(idx, *([t2] * _G_STEP))
    w3 = w_rows.reshape(G_OI, N_LATENT, N_COMP)

    logit_t = pl.pallas_call(
        _logit_body,
        grid=(4,),
        in_specs=[
            pl.BlockSpec((N_LATENT, B), lambda i: (0, 0)),
            pl.BlockSpec((128, N_LATENT, N_COMP), lambda i: (i, 0, 0)),
        ],
        out_specs=pl.BlockSpec((128, N_COMP, B), lambda i: (i, 0, 0)),
        out_shape=jax.ShapeDtypeStruct((G_OI, N_COMP, B), jnp.float32),
    )(latT, w3)

    n_chunks = pl.cdiv(N_GENES, _RHO_CHUNK)
    rho_t = pl.pallas_call(
        _rho_body,
        grid=(n_chunks,),
        in_specs=[
            pl.BlockSpec((N_LATENT, B), lambda i: (0, 0)),
            pl.BlockSpec((N_LATENT, _RHO_CHUNK), lambda i: (0, i)),
        ],
        out_specs=pl.BlockSpec((_RHO_CHUNK, B), lambda i: (i, 0)),
        out_shape=jax.ShapeDtypeStruct((N_GENES, B), jnp.float32),
    )(latT, wT)

    return (jnp.transpose(logit_t, (2, 0, 1)), rho_t.T)


# speedup vs baseline: 8.5303x; 1.0078x over previous
import jax
import jax.numpy as jnp
from jax import lax
from jax.experimental import pallas as pl
from jax.experimental.pallas import tpu as pltpu

N_GENES = 100000
N_LATENT = 16
N_COMP = 32
B = 256
G_OI = 500

_ROW = N_LATENT * N_COMP
_G_STEP = 32
_G_PAD = 512
_RHO_CHUNK = 16384


def _gather_body(idx_ref, *refs):
    bs, out_ref = refs[:-1], refs[-1]
    i = pl.program_id(0)
    lane_iota = lax.broadcasted_iota(jnp.int32, (1, 128), 1)
    for j, blk in enumerate(bs):
        lane = idx_ref[jnp.minimum(_G_STEP * i + j, G_OI - 1)] & 127
        onehot = (lane_iota == lane).astype(jnp.float32)
        out_ref[j, :] = lax.dot_general(
            onehot, blk[...], (((1,), (1,)), ((), ())),
            preferred_element_type=jnp.float32)[0]


def _logit_body(lat_ref, w_ref, out_ref):
    out_ref[...] = lax.dot_general(
        w_ref[...], lat_ref[...], (((1,), (0,)), ((), ())),
        preferred_element_type=jnp.float32)


def _rho_body(lat_ref, w_ref, out_ref):
    out_ref[...] = lax.dot_general(
        w_ref[...], lat_ref[...], (((0,), (0,)), ((), ())),
        preferred_element_type=jnp.float32)


def kernel(latent, genes_oi, logit_weight_table, rho_weight_table):
    latT = latent.T
    wT = rho_weight_table.T
    t2 = jnp.transpose(logit_weight_table, (1, 2, 0)).reshape(_ROW, N_GENES)

    tbl_spec = [
        pl.BlockSpec(
            (_ROW, 128),
            (lambda j: lambda i, idx_ref: (0, idx_ref[jnp.minimum(_G_STEP * i + j, G_OI - 1)] >> 7))(j),
        )
        for j in range(_G_STEP)
    ]
    w_rows = pl.pallas_call(
        _gather_body,
        grid_spec=pltpu.PrefetchScalarGridSpec(
            num_scalar_prefetch=1,
            grid=(_G_PAD // _G_STEP,),
            in_specs=tbl_spec,
            out_specs=pl.BlockSpec((_G_STEP, _ROW), lambda i, idx_ref: (i, 0)),
        ),
        out_shape=jax.ShapeDtypeStruct((G_OI, _ROW), jnp.float32),
    )(genes_oi, *([t2] * _G_STEP))
    w3 = w_rows.reshape(G_OI, N_LATENT, N_COMP)

    logit_t = pl.pallas_call(
        _logit_body,
        grid=(4,),
        in_specs=[
            pl.BlockSpec((N_LATENT, B), lambda i: (0, 0)),
            pl.BlockSpec((128, N_LATENT, N_COMP), lambda i: (i, 0, 0)),
        ],
        out_specs=pl.BlockSpec((128, N_COMP, B), lambda i: (i, 0, 0)),
        out_shape=jax.ShapeDtypeStruct((G_OI, N_COMP, B), jnp.float32),
    )(latT, w3)

    n_chunks = pl.cdiv(N_GENES, _RHO_CHUNK)
    rho_t = pl.pallas_call(
        _rho_body,
        grid=(n_chunks,),
        in_specs=[
            pl.BlockSpec((N_LATENT, B), lambda i: (0, 0)),
            pl.BlockSpec((N_LATENT, _RHO_CHUNK), lambda i: (0, i)),
        ],
        out_specs=pl.BlockSpec((_RHO_CHUNK, B), lambda i: (i, 0)),
        out_shape=jax.ShapeDtypeStruct((N_GENES, B), jnp.float32),
    )(latT, wT)

    return (jnp.transpose(logit_t, (2, 0, 1)), rho_t.T)


# speedup vs baseline: 8.6477x; 1.0138x over previous
import jax
import jax.numpy as jnp
from jax import lax
from jax.experimental import pallas as pl
from jax.experimental.pallas import tpu as pltpu

N_GENES = 100000
N_LATENT = 16
N_COMP = 32
B = 256
G_OI = 500

_ROW = N_LATENT * N_COMP
_G_STEP = 32
_G_PAD = 512
_RHO_CHUNK = 8192


def _gather_body(idx_ref, *refs):
    bs, out_ref = refs[:-1], refs[-1]
    i = pl.program_id(0)
    lane_iota = lax.broadcasted_iota(jnp.int32, (1, 128), 1)
    for j, blk in enumerate(bs):
        lane = idx_ref[jnp.minimum(_G_STEP * i + j, G_OI - 1)] & 127
        onehot = (lane_iota == lane).astype(jnp.float32)
        out_ref[j, :] = lax.dot_general(
            onehot, blk[...], (((1,), (1,)), ((), ())),
            preferred_element_type=jnp.float32)[0]


def _logit_body(lat_ref, w_ref, out_ref):
    out_ref[...] = lax.dot_general(
        w_ref[...], lat_ref[...], (((1,), (0,)), ((), ())),
        preferred_element_type=jnp.float32)


def _rho_body(lat_ref, w_ref, out_ref):
    out_ref[...] = lax.dot_general(
        w_ref[...], lat_ref[...], (((0,), (0,)), ((), ())),
        preferred_element_type=jnp.float32)


def kernel(latent, genes_oi, logit_weight_table, rho_weight_table):
    latT = latent.T
    wT = rho_weight_table.T
    t2 = jnp.transpose(logit_weight_table, (1, 2, 0)).reshape(_ROW, N_GENES)

    tbl_spec = [
        pl.BlockSpec(
            (_ROW, 128),
            (lambda j: lambda i, idx_ref: (0, idx_ref[jnp.minimum(_G_STEP * i + j, G_OI - 1)] >> 7))(j),
        )
        for j in range(_G_STEP)
    ]
    w_rows = pl.pallas_call(
        _gather_body,
        grid_spec=pltpu.PrefetchScalarGridSpec(
            num_scalar_prefetch=1,
            grid=(_G_PAD // _G_STEP,),
            in_specs=tbl_spec,
            out_specs=pl.BlockSpec((_G_STEP, _ROW), lambda i, idx_ref: (i, 0)),
        ),
        out_shape=jax.ShapeDtypeStruct((G_OI, _ROW), jnp.float32),
    )(genes_oi, *([t2] * _G_STEP))
    w3 = w_rows.reshape(G_OI, N_LATENT, N_COMP)

    logit_t = pl.pallas_call(
        _logit_body,
        grid=(4,),
        in_specs=[
            pl.BlockSpec((N_LATENT, B), lambda i: (0, 0)),
            pl.BlockSpec((128, N_LATENT, N_COMP), lambda i: (i, 0, 0)),
        ],
        out_specs=pl.BlockSpec((128, N_COMP, B), lambda i: (i, 0, 0)),
        out_shape=jax.ShapeDtypeStruct((G_OI, N_COMP, B), jnp.float32),
    )(latT, w3)

    n_chunks = pl.cdiv(N_GENES, _RHO_CHUNK)
    rho_t = pl.pallas_call(
        _rho_body,
        grid=(n_chunks,),
        in_specs=[
            pl.BlockSpec((N_LATENT, B), lambda i: (0, 0)),
            pl.BlockSpec((N_LATENT, _RHO_CHUNK), lambda i: (0, i)),
        ],
        out_specs=pl.BlockSpec((_RHO_CHUNK, B), lambda i: (i, 0)),
        out_shape=jax.ShapeDtypeStruct((N_GENES, B), jnp.float32),
    )(latT, wT)

    return (jnp.transpose(logit_t, (2, 0, 1)), rho_t.T)
